# Initial kernel scaffold; baseline (speedup 1.0000x reference)
import jax
import jax.numpy as jnp
from jax.experimental import pallas as pl


def kernel(logits, context_inputs, targets, projection, projection_bias, weights, bias):
    raise NotImplementedError("write your pallas kernel here")


# speedup vs baseline: 58.9415x; 58.9415x over previous
import functools

import jax
import jax.numpy as jnp
from jax.experimental import pallas as pl

SIZE = 1024
INPUT_SIZE = 1024
CONTEXT_SIZE = 512
CMS = 4
NCTX = 2 ** CMS
BATCH = 256
LR = 0.01
OUT_CLIP = 0.01
W_CLIP = 5.0

BS = 64


def _body(bias_ref, logits_ref, ctx_ref, tgt_ref, proj_ref, pbias_ref, w_ref,
          out_ref, wout_ref):
    i = pl.program_id(0)
    f32 = jnp.float32

    proj = proj_ref[...].reshape(BS * CMS, CONTEXT_SIZE).astype(jnp.bfloat16)
    projected = jax.lax.dot_general(
        proj, ctx_ref[...].astype(jnp.bfloat16), (((1,), (0,)), ((), ())),
        preferred_element_type=f32)
    bits = (projected.reshape(BS, CMS, BATCH) > pbias_ref[...]).astype(jnp.int32)
    conv = 1 << jax.lax.broadcasted_iota(jnp.int32, (1, CMS, 1), 1)
    idx = jnp.sum(bits * conv, axis=1)

    w = w_ref[...].reshape(BS * NCTX, INPUT_SIZE)
    all_log = jax.lax.dot_general(
        w.astype(jnp.bfloat16), logits_ref[...].astype(jnp.bfloat16),
        (((1,), (0,)), ((), ())),
        preferred_element_type=f32).reshape(BS, NCTX, BATCH)

    c_iota = jax.lax.broadcasted_iota(jnp.int32, (BS, NCTX, BATCH), 1)
    oh = idx[:, None, :] == c_iota
    out_log = jnp.sum(jnp.where(oh, all_log, 0.0), axis=1)

    s_iota = jax.lax.broadcasted_iota(jnp.int32, (BS, BATCH), 0)
    out_log = jnp.where((i == 0) & (s_iota == 0), bias_ref[0, 0], out_log)
    out_ref[...] = out_log

    sig = jnp.clip(jax.nn.sigmoid(out_log), OUT_CLIP, 1.0 - OUT_CLIP)
    delta = LR * (sig - tgt_ref[...])
    b_iota = jax.lax.broadcasted_iota(jnp.int32, (BS, NCTX, BATCH), 2)
    win = jnp.max(jnp.where(oh, b_iota, -1), axis=2)
    wsel = oh & (b_iota == win[:, :, None])
    wmat = jnp.where(wsel, delta[:, None, :], 0.0).reshape(BS * NCTX, BATCH)
    upd = jax.lax.dot_general(
        wmat, logits_ref[...], (((1,), (1,)), ((), ())),
        preferred_element_type=f32,
        precision=jax.lax.Precision.HIGHEST)
    new_w = jnp.clip(w - upd, -W_CLIP, W_CLIP).reshape(BS, NCTX, INPUT_SIZE)
    has = (win >= 0)[:, :, None]
    wout_ref[...] = jnp.where(has, new_w, w_ref[...])


@functools.partial(jax.jit, static_argnames=("interpret",))
def kernel(logits, context_inputs, targets, projection, projection_bias,
           weights, bias, interpret=False):
    grid = (SIZE // BS,)
    out_log, w_out = pl.pallas_call(
        _body,
        grid=grid,
        in_specs=[
            pl.BlockSpec((1, 1), lambda i: (0, 0)),
            pl.BlockSpec((INPUT_SIZE, BATCH), lambda i: (0, 0)),
            pl.BlockSpec((CONTEXT_SIZE, BATCH), lambda i: (0, 0)),
            pl.BlockSpec((BS, BATCH), lambda i: (i, 0)),
            pl.BlockSpec((BS, CMS, CONTEXT_SIZE), lambda i: (i, 0, 0)),
            pl.BlockSpec((BS, CMS, 1), lambda i: (i, 0, 0)),
            pl.BlockSpec((BS, NCTX, INPUT_SIZE), lambda i: (i, 0, 0)),
        ],
        out_specs=[
            pl.BlockSpec((BS, BATCH), lambda i: (i, 0)),
            pl.BlockSpec((BS, NCTX, INPUT_SIZE), lambda i: (i, 0, 0)),
        ],
        out_shape=[
            jax.ShapeDtypeStruct((SIZE, BATCH), jnp.float32),
            jax.ShapeDtypeStruct((SIZE, NCTX, INPUT_SIZE), jnp.float32),
        ],
        interpret=interpret,
    )(jnp.reshape(bias, (1, 1)), logits, context_inputs, targets,
      projection, projection_bias, weights)
    return out_log, w_out


# speedup vs baseline: 84.6778x; 1.4366x over previous
import functools

import jax
import jax.numpy as jnp
from jax.experimental import pallas as pl

SIZE = 1024
INPUT_SIZE = 1024
CONTEXT_SIZE = 512
CMS = 4
NCTX = 2 ** CMS
BATCH = 256
LR = 0.01
OUT_CLIP = 0.01
W_CLIP = 5.0

BS = 64


def _body(bias_ref, logits_ref, ctx_ref, tgt_ref, proj_ref, pbias_ref, w_ref,
          out_ref, wout_ref):
    i = pl.program_id(0)
    f32 = jnp.float32

    proj = proj_ref[...].reshape(BS * CMS, CONTEXT_SIZE).astype(jnp.bfloat16)
    projected = jax.lax.dot_general(
        proj, ctx_ref[...].astype(jnp.bfloat16), (((1,), (0,)), ((), ())),
        preferred_element_type=f32)
    bits = (projected.reshape(BS, CMS, BATCH) > pbias_ref[...]).astype(jnp.int32)
    conv = 1 << jax.lax.broadcasted_iota(jnp.int32, (1, CMS, 1), 1)
    idx = jnp.sum(bits * conv, axis=1)

    w = w_ref[...].reshape(BS * NCTX, INPUT_SIZE)
    all_log = jax.lax.dot_general(
        w.astype(jnp.bfloat16), logits_ref[...].astype(jnp.bfloat16),
        (((1,), (0,)), ((), ())),
        preferred_element_type=f32).reshape(BS, NCTX, BATCH)

    c_iota = jax.lax.broadcasted_iota(jnp.int32, (BS, NCTX, BATCH), 1)
    oh = idx[:, None, :] == c_iota
    out_log = jnp.sum(jnp.where(oh, all_log, 0.0), axis=1)

    s_iota = jax.lax.broadcasted_iota(jnp.int32, (BS, BATCH), 0)
    out_log = jnp.where((i == 0) & (s_iota == 0), bias_ref[0, 0], out_log)
    out_ref[...] = out_log

    sig = jnp.clip(jax.nn.sigmoid(out_log), OUT_CLIP, 1.0 - OUT_CLIP)
    delta = LR * (sig - tgt_ref[...])
    b_iota = jax.lax.broadcasted_iota(jnp.int32, (BS, NCTX, BATCH), 2)
    win = jnp.max(jnp.where(oh, b_iota, -1), axis=2)
    wsel = oh & (b_iota == win[:, :, None])
    wmat = jnp.where(wsel, delta[:, None, :], 0.0).reshape(BS * NCTX, BATCH)
    upd = jax.lax.dot_general(
        wmat.astype(jnp.bfloat16), logits_ref[...].astype(jnp.bfloat16),
        (((1,), (1,)), ((), ())),
        preferred_element_type=f32)
    new_w = jnp.clip(w - upd, -W_CLIP, W_CLIP).reshape(BS, NCTX, INPUT_SIZE)
    has = (win >= 0)[:, :, None]
    wout_ref[...] = jnp.where(has, new_w, w_ref[...])


@functools.partial(jax.jit, static_argnames=("interpret",))
def kernel(logits, context_inputs, targets, projection, projection_bias,
           weights, bias, interpret=False):
    grid = (SIZE // BS,)
    out_log, w_out = pl.pallas_call(
        _body,
        grid=grid,
        in_specs=[
            pl.BlockSpec((1, 1), lambda i: (0, 0)),
            pl.BlockSpec((INPUT_SIZE, BATCH), lambda i: (0, 0)),
            pl.BlockSpec((CONTEXT_SIZE, BATCH), lambda i: (0, 0)),
            pl.BlockSpec((BS, BATCH), lambda i: (i, 0)),
            pl.BlockSpec((BS, CMS, CONTEXT_SIZE), lambda i: (i, 0, 0)),
            pl.BlockSpec((BS, CMS, 1), lambda i: (i, 0, 0)),
            pl.BlockSpec((BS, NCTX, INPUT_SIZE), lambda i: (i, 0, 0)),
        ],
        out_specs=[
            pl.BlockSpec((BS, BATCH), lambda i: (i, 0)),
            pl.BlockSpec((BS, NCTX, INPUT_SIZE), lambda i: (i, 0, 0)),
        ],
        out_shape=[
            jax.ShapeDtypeStruct((SIZE, BATCH), jnp.float32),
            jax.ShapeDtypeStruct((SIZE, NCTX, INPUT_SIZE), jnp.float32),
        ],
        interpret=interpret,
    )(jnp.reshape(bias, (1, 1)), logits, context_inputs, targets,
      projection, projection_bias, weights)
    return out_log, w_out
